# baseline (device time: 22108 ns/iter reference)
import jax
import jax.numpy as jnp
from jax import lax
from jax.experimental import pallas as pl
from jax.experimental.pallas import tpu as pltpu

N_DEV = 4
N_TOK = 512
D_IN = 256
D_OUT = 512
N_EXP = 8
EXP_PER_DEV = N_EXP // N_DEV
ROWS = N_TOK // N_DEV


def kernel(x, router_W, route_idx, expert_W):
    def body(x_ref, rw_ref, idx_ref, ew_ref, out_ref,
             acc_ref, comm_ref, send_sems, recv_sems):
        my = lax.axis_index("i")
        left = (my + N_DEV - 1) % N_DEV
        right = (my + 1) % N_DEV

        barrier_sem = pltpu.get_barrier_semaphore()
        for nbr in (left, right):
            pl.semaphore_signal(
                barrier_sem, inc=1,
                device_id=(nbr,), device_id_type=pl.DeviceIdType.MESH,
            )
        pl.semaphore_wait(barrier_sem, 2)

        xv = x_ref[:, :]
        scores = jnp.dot(xv, rw_ref[:, :], preferred_element_type=jnp.float32)
        m = jnp.max(scores, axis=-1, keepdims=True)
        p = jnp.exp(scores - m)
        probs = p / jnp.sum(p, axis=-1, keepdims=True)

        idx0 = idx_ref[:, 0:1]
        idx1 = idx_ref[:, 1:2]
        iota = lax.broadcasted_iota(jnp.int32, (N_TOK, N_EXP), 1)
        sel = (iota == idx0) | (iota == idx1)
        gs = jnp.sum(jnp.where(sel, probs, 0.0), axis=-1, keepdims=True)

        partial = jnp.zeros((N_TOK, D_OUT), jnp.float32)
        for le in range(EXP_PER_DEV):
            e = my * EXP_PER_DEV + le
            gate = jnp.sum(
                jnp.where(sel & (iota == e), probs, 0.0),
                axis=-1, keepdims=True,
            ) / gs
            xg = (xv * gate).astype(jnp.bfloat16)
            partial = partial + jnp.dot(
                xg, ew_ref[le].astype(jnp.bfloat16),
                preferred_element_type=jnp.float32,
            )

        for k in range(N_DEV):
            acc_ref[k] = partial[k * ROWS:(k + 1) * ROWS, :]

        for s in range(N_DEV - 1):
            send_chunk = (my + N_DEV - 1 - s) % N_DEV
            recv_chunk = (my + N_DEV - 2 - s) % N_DEV
            rdma = pltpu.make_async_remote_copy(
                src_ref=acc_ref.at[send_chunk],
                dst_ref=comm_ref.at[s],
                send_sem=send_sems.at[s],
                recv_sem=recv_sems.at[s],
                device_id=(right,),
                device_id_type=pl.DeviceIdType.MESH,
            )
            rdma.start()
            rdma.wait()
            acc_ref[recv_chunk] = acc_ref[recv_chunk] + comm_ref[s]

        out_ref[:, :] = acc_ref[my]

    return pl.pallas_call(
        body,
        out_shape=jax.ShapeDtypeStruct((ROWS, D_OUT), jnp.float32),
        in_specs=[pl.BlockSpec(memory_space=pltpu.VMEM)] * 4,
        out_specs=pl.BlockSpec(memory_space=pltpu.VMEM),
        scratch_shapes=[
            pltpu.VMEM((N_DEV, ROWS, D_OUT), jnp.float32),
            pltpu.VMEM((N_DEV - 1, ROWS, D_OUT), jnp.float32),
            pltpu.SemaphoreType.DMA((N_DEV - 1,)),
            pltpu.SemaphoreType.DMA((N_DEV - 1,)),
        ],
        compiler_params=pltpu.CompilerParams(collective_id=0),
    )(x, router_W, route_idx, expert_W)


# device time: 12869 ns/iter; 1.7179x vs baseline; 1.7179x over previous
import jax
import jax.numpy as jnp
from jax import lax
from jax.experimental import pallas as pl
from jax.experimental.pallas import tpu as pltpu

N_DEV = 4
N_TOK = 512
D_IN = 256
D_OUT = 512
N_EXP = 8
EXP_PER_DEV = N_EXP // N_DEV
ROWS = N_TOK // N_DEV


def kernel(x, router_W, route_idx, expert_W):
    def body(x_ref, rw_ref, idx_ref, ew_ref, out_ref,
             xg_ref, send_ref, comm_ref, send_sems, recv_sems):
        my = lax.axis_index("i")

        barrier_sem = pltpu.get_barrier_semaphore()
        for j in range(1, N_DEV):
            pl.semaphore_signal(
                barrier_sem, inc=1,
                device_id=((my + j) % N_DEV,),
                device_id_type=pl.DeviceIdType.MESH,
            )
        pl.semaphore_wait(barrier_sem, N_DEV - 1)

        xv = x_ref[:, :]
        scores = jnp.dot(xv, rw_ref[:, :], preferred_element_type=jnp.float32)
        mx = jnp.max(scores, axis=-1, keepdims=True)
        p = jnp.exp(scores - mx)
        probs = p / jnp.sum(p, axis=-1, keepdims=True)

        idx0 = idx_ref[:, 0:1]
        idx1 = idx_ref[:, 1:2]
        iota = lax.broadcasted_iota(jnp.int32, (N_TOK, N_EXP), 1)
        sel = (iota == idx0) | (iota == idx1)
        gs = jnp.sum(jnp.where(sel, probs, 0.0), axis=-1, keepdims=True)

        for le in range(EXP_PER_DEV):
            e = my * EXP_PER_DEV + le
            gate = jnp.sum(
                jnp.where(sel & (iota == e), probs, 0.0),
                axis=-1, keepdims=True,
            ) / gs
            xg_ref[le] = (xv * gate).astype(jnp.bfloat16)

        w0 = ew_ref[0].astype(jnp.bfloat16)
        w1 = ew_ref[1].astype(jnp.bfloat16)

        def chunk_partial(c):
            a = xg_ref[0, pl.ds(c * ROWS, ROWS), :]
            b = xg_ref[1, pl.ds(c * ROWS, ROWS), :]
            return (jnp.dot(a, w0, preferred_element_type=jnp.float32)
                    + jnp.dot(b, w1, preferred_element_type=jnp.float32))

        rdmas = []
        for j in (1, 0, 2):
            d = (my + 1 + j) % N_DEV
            send_ref[j] = chunk_partial(d).astype(jnp.bfloat16)
            rdma = pltpu.make_async_remote_copy(
                src_ref=send_ref.at[j],
                dst_ref=comm_ref.at[2 - j],
                send_sem=send_sems.at[j],
                recv_sem=recv_sems.at[2 - j],
                device_id=(d,),
                device_id_type=pl.DeviceIdType.MESH,
            )
            rdma.start()
            rdmas.append(rdma)

        own = chunk_partial(my)

        for rdma in rdmas:
            rdma.wait_recv()
        acc = own
        for s in range(N_DEV - 1):
            acc = acc + comm_ref[s].astype(jnp.float32)
        out_ref[:, :] = acc

        for rdma in rdmas:
            rdma.wait_send()

    return pl.pallas_call(
        body,
        out_shape=jax.ShapeDtypeStruct((ROWS, D_OUT), jnp.float32),
        in_specs=[pl.BlockSpec(memory_space=pltpu.VMEM)] * 4,
        out_specs=pl.BlockSpec(memory_space=pltpu.VMEM),
        scratch_shapes=[
            pltpu.VMEM((EXP_PER_DEV, N_TOK, D_IN), jnp.bfloat16),
            pltpu.VMEM((N_DEV - 1, ROWS, D_OUT), jnp.bfloat16),
            pltpu.VMEM((N_DEV - 1, ROWS, D_OUT), jnp.bfloat16),
            pltpu.SemaphoreType.DMA((N_DEV - 1,)),
            pltpu.SemaphoreType.DMA((N_DEV - 1,)),
        ],
        compiler_params=pltpu.CompilerParams(collective_id=0),
    )(x, router_W, route_idx, expert_W)


# device time: 12456 ns/iter; 1.7749x vs baseline; 1.0332x over previous
import jax
import jax.numpy as jnp
from jax import lax
from jax.experimental import pallas as pl
from jax.experimental.pallas import tpu as pltpu

N_DEV = 4
N_TOK = 512
D_IN = 256
D_OUT = 512
N_EXP = 8
EXP_PER_DEV = N_EXP // N_DEV
ROWS = N_TOK // N_DEV
K_CAT = EXP_PER_DEV * D_IN


def kernel(x, router_W, route_idx, expert_W):
    def body(x_ref, rw_ref, idx_ref, ew_ref, out_ref,
             xg_ref, send_ref, comm_ref, send_sems, recv_sems):
        my = lax.axis_index("i")

        barrier_sem = pltpu.get_barrier_semaphore()
        for j in range(1, N_DEV):
            pl.semaphore_signal(
                barrier_sem, inc=1,
                device_id=((my + j) % N_DEV,),
                device_id_type=pl.DeviceIdType.MESH,
            )

        xv = x_ref[:, :]
        scores = jnp.dot(xv, rw_ref[:, :], preferred_element_type=jnp.float32)
        mx = jnp.max(scores, axis=-1, keepdims=True)
        p = jnp.exp(scores - mx)
        probs = p / jnp.sum(p, axis=-1, keepdims=True)

        idx0 = idx_ref[:, 0:1]
        idx1 = idx_ref[:, 1:2]
        iota = lax.broadcasted_iota(jnp.int32, (N_TOK, N_EXP), 1)
        sel = (iota == idx0) | (iota == idx1)
        gs = jnp.sum(jnp.where(sel, probs, 0.0), axis=-1, keepdims=True)

        for le in range(EXP_PER_DEV):
            e = my * EXP_PER_DEV + le
            gate = jnp.sum(
                jnp.where(sel & (iota == e), probs, 0.0),
                axis=-1, keepdims=True,
            ) / gs
            xg_ref[:, le * D_IN:(le + 1) * D_IN] = (xv * gate).astype(
                jnp.bfloat16)

        w_cat = jnp.reshape(
            ew_ref[:, :, :], (K_CAT, D_OUT)).astype(jnp.bfloat16)

        def chunk_partial(c):
            return jnp.dot(
                xg_ref[pl.ds(c * ROWS, ROWS), :], w_cat,
                preferred_element_type=jnp.float32)

        pl.semaphore_wait(barrier_sem, N_DEV - 1)

        rdmas = {}
        for j in (1, 0, 2):
            d = (my + 1 + j) % N_DEV
            send_ref[j] = chunk_partial(d).astype(jnp.bfloat16)
            rdmas[j] = pltpu.make_async_remote_copy(
                src_ref=send_ref.at[j],
                dst_ref=comm_ref.at[2 - j],
                send_sem=send_sems.at[j],
                recv_sem=recv_sems.at[2 - j],
                device_id=(d,),
                device_id_type=pl.DeviceIdType.MESH,
            )
            rdmas[j].start()

        acc = chunk_partial(my)

        for j in (0, 2, 1):
            rdmas[j].wait_recv()
            acc = acc + comm_ref[2 - j].astype(jnp.float32)
        out_ref[:, :] = acc

        for j in (1, 0, 2):
            rdmas[j].wait_send()

    return pl.pallas_call(
        body,
        out_shape=jax.ShapeDtypeStruct((ROWS, D_OUT), jnp.float32),
        in_specs=[pl.BlockSpec(memory_space=pltpu.VMEM)] * 4,
        out_specs=pl.BlockSpec(memory_space=pltpu.VMEM),
        scratch_shapes=[
            pltpu.VMEM((N_TOK, K_CAT), jnp.bfloat16),
            pltpu.VMEM((N_DEV - 1, ROWS, D_OUT), jnp.bfloat16),
            pltpu.VMEM((N_DEV - 1, ROWS, D_OUT), jnp.bfloat16),
            pltpu.SemaphoreType.DMA((N_DEV - 1,)),
            pltpu.SemaphoreType.DMA((N_DEV - 1,)),
        ],
        compiler_params=pltpu.CompilerParams(collective_id=0),
    )(x, router_W, route_idx, expert_W)


# device time: 10561 ns/iter; 2.0934x vs baseline; 1.1794x over previous
import jax
import jax.numpy as jnp
from jax import lax
from jax.experimental import pallas as pl
from jax.experimental.pallas import tpu as pltpu

N_DEV = 4
N_TOK = 512
D_IN = 256
D_OUT = 512
N_EXP = 8
EXP_PER_DEV = N_EXP // N_DEV
ROWS = N_TOK // N_DEV
K_CAT = EXP_PER_DEV * D_IN


def _pallas_moe_rs(x, rwt, idxt, ew):
    def body(x_hbm, rwt_hbm, idxt_hbm, ew_hbm, out_ref,
             xv_ref, rwt_ref, idxt_ref, ew_ref, xg_ref, send_ref, comm_ref,
             copy_sems, send_sems, recv_sems,
             gate_from_left, gate_from_right):
        my = lax.axis_index("i")
        right = (my + 1) % N_DEV
        diag = (my + 2) % N_DEV
        left = (my + 3) % N_DEV

        cp_x = pltpu.make_async_copy(x_hbm, xv_ref, copy_sems.at[0])
        cp_ew = pltpu.make_async_copy(ew_hbm, ew_ref, copy_sems.at[1])
        cp_rw = pltpu.make_async_copy(rwt_hbm, rwt_ref, copy_sems.at[2])
        cp_idx = pltpu.make_async_copy(idxt_hbm, idxt_ref, copy_sems.at[3])
        cp_x.start()
        cp_ew.start()
        cp_rw.start()
        cp_idx.start()

        barrier_sem = pltpu.get_barrier_semaphore()
        pl.semaphore_signal(
            barrier_sem, inc=1,
            device_id=(diag,), device_id_type=pl.DeviceIdType.MESH)
        pl.semaphore_signal(
            gate_from_left, inc=1,
            device_id=(right,), device_id_type=pl.DeviceIdType.MESH)
        pl.semaphore_signal(
            gate_from_right, inc=1,
            device_id=(left,), device_id_type=pl.DeviceIdType.MESH)

        cp_x.wait()
        cp_rw.wait()
        cp_idx.wait()
        xv = xv_ref[:, :]
        rwb = rwt_ref[:, :].astype(jnp.bfloat16)
        scores = lax.dot_general(
            xv.astype(jnp.bfloat16), rwb, (((1,), (1,)), ((), ())),
            preferred_element_type=jnp.float32)
        mx = jnp.max(scores, axis=-1, keepdims=True)
        p = jnp.exp(scores - mx)
        probs = p / jnp.sum(p, axis=-1, keepdims=True)

        idx0 = jnp.transpose(idxt_ref[0:1, :])
        idx1 = jnp.transpose(idxt_ref[1:2, :])
        iota = lax.broadcasted_iota(jnp.int32, (N_TOK, N_EXP), 1)
        sel = (iota == idx0) | (iota == idx1)
        gs = jnp.sum(jnp.where(sel, probs, 0.0), axis=-1, keepdims=True)

        for le in range(EXP_PER_DEV):
            e = my * EXP_PER_DEV + le
            gate = jnp.sum(
                jnp.where(sel & (iota == e), probs, 0.0),
                axis=-1, keepdims=True,
            ) / gs
            xg_ref[:, le * D_IN:(le + 1) * D_IN] = (xv * gate).astype(
                jnp.bfloat16)

        cp_ew.wait()
        w = jnp.reshape(ew_ref[:, :, :], (K_CAT, D_OUT)).astype(jnp.bfloat16)

        def chunk_partial(c):
            return jnp.dot(
                xg_ref[pl.ds(c * ROWS, ROWS), :], w,
                preferred_element_type=jnp.float32)

        gates = {1: barrier_sem, 0: gate_from_right, 2: gate_from_left}
        for j in (1, 0, 2):
            d = (my + 1 + j) % N_DEV
            send_ref[j] = chunk_partial(d).astype(jnp.bfloat16)
        rdmas = {}
        for j in (1, 0, 2):
            d = (my + 1 + j) % N_DEV
            rdmas[j] = pltpu.make_async_remote_copy(
                src_ref=send_ref.at[j],
                dst_ref=comm_ref.at[2 - j],
                send_sem=send_sems.at[j],
                recv_sem=recv_sems.at[2 - j],
                device_id=(d,),
                device_id_type=pl.DeviceIdType.MESH,
            )
            pl.semaphore_wait(gates[j], 1)
            rdmas[j].start()

        acc = chunk_partial(my)

        for j in (0, 2, 1):
            rdmas[j].wait_recv()
            acc = acc + comm_ref[2 - j].astype(jnp.float32)
        out_ref[:, :] = acc

        for j in (1, 0, 2):
            rdmas[j].wait_send()

    return pl.pallas_call(
        body,
        out_shape=jax.ShapeDtypeStruct((ROWS, D_OUT), jnp.float32),
        in_specs=[pl.BlockSpec(memory_space=pl.ANY)] * 4,
        out_specs=pl.BlockSpec(memory_space=pltpu.VMEM),
        scratch_shapes=[
            pltpu.VMEM((N_TOK, D_IN), jnp.float32),
            pltpu.VMEM((N_EXP, D_IN), jnp.float32),
            pltpu.VMEM((2, N_TOK), jnp.int32),
            pltpu.VMEM((EXP_PER_DEV, D_IN, D_OUT), jnp.float32),
            pltpu.VMEM((N_TOK, K_CAT), jnp.bfloat16),
            pltpu.VMEM((N_DEV - 1, ROWS, D_OUT), jnp.bfloat16),
            pltpu.VMEM((N_DEV - 1, ROWS, D_OUT), jnp.bfloat16),
            pltpu.SemaphoreType.DMA((4,)),
            pltpu.SemaphoreType.DMA((N_DEV - 1,)),
            pltpu.SemaphoreType.DMA((N_DEV - 1,)),
            pltpu.SemaphoreType.REGULAR,
            pltpu.SemaphoreType.REGULAR,
        ],
        compiler_params=pltpu.CompilerParams(collective_id=0),
    )(
        pltpu.with_memory_space_constraint(x, pltpu.MemorySpace.HBM),
        pltpu.with_memory_space_constraint(rwt, pltpu.MemorySpace.HBM),
        pltpu.with_memory_space_constraint(idxt, pltpu.MemorySpace.HBM),
        pltpu.with_memory_space_constraint(ew, pltpu.MemorySpace.HBM),
    )


def kernel(x, router_W, route_idx, expert_W):
    rwt = jnp.transpose(router_W)
    idxt = jnp.transpose(route_idx)
    return _pallas_moe_rs(x, rwt, idxt, expert_W)
